# Initial kernel scaffold; baseline (speedup 1.0000x reference)
import jax
import jax.numpy as jnp
from jax.experimental import pallas as pl


def kernel(x, edge_index, Wq, bq, Wk, bk, Wv, bv, Wo, bo, gamma, beta):
    raise NotImplementedError("write your pallas kernel here")


# speedup vs baseline: 1.6991x; 1.6991x over previous
import functools
import math

import jax
import jax.numpy as jnp
from jax.experimental import pallas as pl
from jax.experimental.pallas import tpu as pltpu

N = 2048
D = 256
H = 8
HD = D // H
BQ = 256
GRID = N // BQ
RSCALE = 1.0 / math.sqrt(HD)


def _attn_body(x_blk, x_full, m_ref, wqT, bq, wkT, bk, wvT, bv, woT, bo,
               gamma, beta, out_ref, k_scr, v_scr):
    i = pl.program_id(0)

    @pl.when(i == 0)
    def _():
        xb = x_full[...].astype(jnp.bfloat16)
        k_scr[...] = (jnp.dot(xb, wkT[...], preferred_element_type=jnp.float32)
                      + bk[...]).astype(jnp.bfloat16)
        v_scr[...] = (jnp.dot(xb, wvT[...], preferred_element_type=jnp.float32)
                      + bv[...]).astype(jnp.bfloat16)

    xq = x_blk[...]
    q = jnp.dot(xq.astype(jnp.bfloat16), wqT[...],
                preferred_element_type=jnp.float32) + bq[...]
    q = q.astype(jnp.bfloat16)

    mi = m_ref[pl.ds(i * BQ, BQ), :]
    reach = jnp.dot(mi, m_ref[...], preferred_element_type=jnp.float32)
    maskb = reach > 0.0

    kb = k_scr[...]
    vb = v_scr[...]

    att_cols = []
    for h in range(H):
        sl = slice(h * HD, (h + 1) * HD)
        qh = q[:, sl]
        kh = kb[:, sl]
        s = jax.lax.dot_general(qh, kh, (((1,), (1,)), ((), ())),
                                preferred_element_type=jnp.float32)
        e = jnp.where(maskb, jnp.exp(s * RSCALE), 0.0)
        denom = jnp.sum(e, axis=1, keepdims=True)
        vh = vb[:, sl]
        ah = jax.lax.dot_general(e.astype(jnp.bfloat16), vh,
                                 (((1,), (0,)), ((), ())),
                                 preferred_element_type=jnp.float32)
        att_cols.append(ah / denom)
    att = jnp.concatenate(att_cols, axis=1)

    out = jnp.dot(att.astype(jnp.bfloat16), woT[...],
                  preferred_element_type=jnp.float32) + bo[...]
    y = out + xq
    mu = jnp.mean(y, axis=1, keepdims=True)
    yc = y - mu
    var = jnp.mean(yc * yc, axis=1, keepdims=True)
    out_ref[...] = yc * jax.lax.rsqrt(var + 1e-5) * gamma[...] + beta[...]


@jax.jit
def _attn_call(x, m, wqT, bq, wkT, bk, wvT, bv, woT, bo, gamma, beta):
    full = lambda *_: (0, 0)
    specs = [
        pl.BlockSpec((BQ, D), lambda i: (i, 0)),
        pl.BlockSpec((N, D), full),
        pl.BlockSpec((N, N), full),
        pl.BlockSpec((D, D), full),
        pl.BlockSpec((1, D), full),
        pl.BlockSpec((D, D), full),
        pl.BlockSpec((1, D), full),
        pl.BlockSpec((D, D), full),
        pl.BlockSpec((1, D), full),
        pl.BlockSpec((D, D), full),
        pl.BlockSpec((1, D), full),
        pl.BlockSpec((1, D), full),
        pl.BlockSpec((1, D), full),
    ]
    return pl.pallas_call(
        _attn_body,
        grid=(GRID,),
        in_specs=specs,
        out_specs=pl.BlockSpec((BQ, D), lambda i: (i, 0)),
        out_shape=jax.ShapeDtypeStruct((N, D), jnp.float32),
        scratch_shapes=[
            pltpu.VMEM((N, D), jnp.bfloat16),
            pltpu.VMEM((N, D), jnp.bfloat16),
        ],
    )(x, x, m, wqT, bq, wkT, bk, wvT, bv, woT, bo, gamma, beta)


def _build_m(edge_index):
    src, dst = edge_index[0], edge_index[1]
    m = jnp.zeros((N, N), jnp.float32)
    m = m.at[src, dst].add(1.0)
    m = m.at[dst, src].add(1.0)
    diag = jnp.arange(N, dtype=jnp.int32)
    m = m.at[diag, diag].add(1.0)
    return m.astype(jnp.bfloat16)


def kernel(x, edge_index, Wq, bq, Wk, bk, Wv, bv, Wo, bo, gamma, beta):
    m = _build_m(edge_index)
    r = lambda b: b.reshape(1, D)
    return _attn_call(
        x, m,
        Wq.T.astype(jnp.bfloat16), r(bq),
        Wk.T.astype(jnp.bfloat16), r(bk),
        Wv.T.astype(jnp.bfloat16), r(bv),
        Wo.T.astype(jnp.bfloat16), r(bo),
        r(gamma), r(beta))


# speedup vs baseline: 3.0622x; 1.8022x over previous
import functools
import math

import jax
import jax.numpy as jnp
from jax.experimental import pallas as pl
from jax.experimental.pallas import tpu as pltpu
from jax.experimental.pallas import tpu_sc as plsc

N = 2048
D = 256
H = 8
HD = D // H
BQ = 256
GRID = N // BQ
RSCALE = 1.0 / math.sqrt(HD)


def _attn_body(x_blk, x_full, m_ref, wqT, bq, wkT, bk, wvT, bv, woT, bo,
               gamma, beta, out_ref, k_scr, v_scr, m16):
    i = pl.program_id(0)

    @pl.when(i == 0)
    def _():
        xb = x_full[...].astype(jnp.bfloat16)
        k_scr[...] = (jnp.dot(xb, wkT[...], preferred_element_type=jnp.float32)
                      + bk[...]).astype(jnp.bfloat16)
        v_scr[...] = (jnp.dot(xb, wvT[...], preferred_element_type=jnp.float32)
                      + bv[...]).astype(jnp.bfloat16)
        m16[...] = m_ref[...].astype(jnp.bfloat16)

    xq = x_blk[...]
    q = jnp.dot(xq.astype(jnp.bfloat16), wqT[...],
                preferred_element_type=jnp.float32) + bq[...]
    q = q.astype(jnp.bfloat16)

    mi = m16[pl.ds(i * BQ, BQ), :]
    reach = jnp.dot(mi, m16[...], preferred_element_type=jnp.float32)
    maskb = reach > 0.0

    kb = k_scr[...]
    vb = v_scr[...]

    att_cols = []
    for h in range(H):
        sl = slice(h * HD, (h + 1) * HD)
        qh = q[:, sl]
        kh = kb[:, sl]
        s = jax.lax.dot_general(qh, kh, (((1,), (1,)), ((), ())),
                                preferred_element_type=jnp.float32)
        e = jnp.where(maskb, jnp.exp(s * RSCALE), 0.0)
        denom = jnp.sum(e, axis=1, keepdims=True)
        vh = vb[:, sl]
        ah = jax.lax.dot_general(e.astype(jnp.bfloat16), vh,
                                 (((1,), (0,)), ((), ())),
                                 preferred_element_type=jnp.float32)
        att_cols.append(ah / denom)
    att = jnp.concatenate(att_cols, axis=1)

    out = jnp.dot(att.astype(jnp.bfloat16), woT[...],
                  preferred_element_type=jnp.float32) + bo[...]
    y = out + xq
    mu = jnp.mean(y, axis=1, keepdims=True)
    yc = y - mu
    var = jnp.mean(yc * yc, axis=1, keepdims=True)
    out_ref[...] = yc * jax.lax.rsqrt(var + 1e-5) * gamma[...] + beta[...]


@jax.jit
def _attn_call(x, m, wqT, bq, wkT, bk, wvT, bv, woT, bo, gamma, beta):
    full = lambda *_: (0, 0)
    specs = [
        pl.BlockSpec((BQ, D), lambda i: (i, 0)),
        pl.BlockSpec((N, D), full),
        pl.BlockSpec((N, N), full),
        pl.BlockSpec((D, D), full),
        pl.BlockSpec((1, D), full),
        pl.BlockSpec((D, D), full),
        pl.BlockSpec((1, D), full),
        pl.BlockSpec((D, D), full),
        pl.BlockSpec((1, D), full),
        pl.BlockSpec((D, D), full),
        pl.BlockSpec((1, D), full),
        pl.BlockSpec((1, D), full),
        pl.BlockSpec((1, D), full),
    ]
    return pl.pallas_call(
        _attn_body,
        grid=(GRID,),
        in_specs=specs,
        out_specs=pl.BlockSpec((BQ, D), lambda i: (i, 0)),
        out_shape=jax.ShapeDtypeStruct((N, D), jnp.float32),
        scratch_shapes=[
            pltpu.VMEM((N, D), jnp.bfloat16),
            pltpu.VMEM((N, D), jnp.bfloat16),
            pltpu.VMEM((N, N), jnp.bfloat16),
        ],
    )(x, x, m, wqT, bq, wkT, bk, wvT, bv, woT, bo, gamma, beta)


E = 32768
NW = 32
RPB = 32
ECH = 16384


def _sc_scatter_body(edges, out_hbm, rows_v, sbuf, dbuf):
    wid = jax.lax.axis_index("s") * 2 + jax.lax.axis_index("c")
    lanes = jax.lax.iota(jnp.int32, 16)
    ones = jnp.ones((16,), jnp.float32)
    zeros = jnp.zeros((16,), jnp.float32)

    for blk in range(2):
        row_base = (wid * 2 + blk) * RPB

        def _zero(r, c):
            for k in range(N // 16):
                rows_v[r, pl.ds(k * 16, 16)] = zeros
            return c
        jax.lax.fori_loop(0, RPB, _zero, 0)

        for k in range(2):
            r = lanes + k * 16
            plsc.store_scatter(rows_v, [r, r + row_base], ones)

        for half in range(2):
            pltpu.sync_copy(edges.at[pl.ds(half * ECH, ECH)], sbuf)
            pltpu.sync_copy(edges.at[pl.ds(E + half * ECH, ECH)], dbuf)

            def _scan(j, c):
                s = sbuf[pl.ds(j * 16, 16)]
                d = dbuf[pl.ds(j * 16, 16)]
                rel = s - row_base
                own = (rel >= 0) & (rel < RPB)
                plsc.store_scatter(rows_v, [jnp.where(own, rel, 0), d],
                                   ones, mask=own)
                rel2 = d - row_base
                own2 = (rel2 >= 0) & (rel2 < RPB)
                plsc.store_scatter(rows_v, [jnp.where(own2, rel2, 0), s],
                                   ones, mask=own2)
                return c
            jax.lax.fori_loop(0, ECH // 16, _scan, 0)

        pltpu.sync_copy(rows_v, out_hbm.at[pl.ds(row_base, RPB), :])


@jax.jit
def _build_m(edge_index):
    edges = edge_index.reshape(2 * E)
    call = pl.kernel(
        _sc_scatter_body,
        out_type=jax.ShapeDtypeStruct((N, N), jnp.float32),
        mesh=plsc.VectorSubcoreMesh(core_axis_name="c", subcore_axis_name="s"),
        compiler_params=pltpu.CompilerParams(needs_layout_passes=False),
        scratch_types=[
            pltpu.VMEM((RPB, N), jnp.float32),
            pltpu.VMEM((ECH,), jnp.int32),
            pltpu.VMEM((ECH,), jnp.int32),
        ],
    )
    return call(edges)


def kernel(x, edge_index, Wq, bq, Wk, bk, Wv, bv, Wo, bo, gamma, beta):
    m = _build_m(edge_index.astype(jnp.int32))
    r = lambda b: b.reshape(1, D)
    return _attn_call(
        x, m,
        Wq.T.astype(jnp.bfloat16), r(bq),
        Wk.T.astype(jnp.bfloat16), r(bk),
        Wv.T.astype(jnp.bfloat16), r(bv),
        Wo.T.astype(jnp.bfloat16), r(bo),
        r(gamma), r(beta))


# speedup vs baseline: 3.6040x; 1.1769x over previous
import functools
import math

import jax
import jax.numpy as jnp
from jax.experimental import pallas as pl
from jax.experimental.pallas import tpu as pltpu
from jax.experimental.pallas import tpu_sc as plsc

N = 2048
D = 256
H = 8
HD = D // H
BQ = 256
GRID = N // BQ
RSCALE = 1.0 / math.sqrt(HD)
VR = 40


def _attn_body(x_blk, x_full, m_ref, wq, bq, wk, bk, wv, bv, woT, bo,
               gamma, beta, out_ref, kt_scr, vt_scr, m8):
    i = pl.program_id(0)
    f32 = jnp.float32
    bf16 = jnp.bfloat16

    @pl.when(i == 0)
    def _():
        xb = x_full[...].astype(bf16)
        kt_scr[...] = (jax.lax.dot_general(
            wk[...], xb, (((1,), (1,)), ((), ())),
            preferred_element_type=f32) + bk[...]).astype(bf16)
        vt = (jax.lax.dot_general(
            wv[...], xb, (((1,), (1,)), ((), ())),
            preferred_element_type=f32) + bv[...]).astype(bf16)
        for h in range(H):
            vt_scr[h * VR:h * VR + HD, :] = vt[h * HD:(h + 1) * HD, :]
            vt_scr[h * VR + HD:h * VR + HD + 1, :] = jnp.ones((1, N), bf16)
            vt_scr[h * VR + HD + 1:(h + 1) * VR, :] = (
                jnp.zeros((VR - HD - 1, N), bf16))
        m8[...] = m_ref[...].astype(jnp.float8_e4m3fn)

    xq = x_blk[...]
    qt = ((jax.lax.dot_general(wq[...], xq.astype(bf16),
                               (((1,), (1,)), ((), ())),
                               preferred_element_type=f32)
           + bq[...]) * RSCALE).astype(bf16)

    mi = m8[pl.ds(i * BQ, BQ), :]
    reach = jax.lax.dot_general(mi, m8[...], (((1,), (0,)), ((), ())),
                                preferred_element_type=f32)
    nz16 = jnp.minimum(reach, 1.0).astype(bf16)

    att_rows = []
    for h in range(H):
        qh = qt[h * HD:(h + 1) * HD, :]
        kh = kt_scr[h * HD:(h + 1) * HD, :]
        s = jax.lax.dot_general(qh, kh, (((0,), (0,)), ((), ())),
                                preferred_element_type=f32)
        e16 = jnp.exp(s.astype(bf16)) * nz16
        vh = vt_scr[h * VR:h * VR + HD + 1, :]
        aT = jax.lax.dot_general(vh, e16, (((1,), (1,)), ((), ())),
                                 preferred_element_type=f32)
        att_rows.append(aT[:HD, :] / aT[HD:HD + 1, :])
    attT = jnp.concatenate(att_rows, axis=0)

    out = jax.lax.dot_general(attT.astype(bf16), woT[...],
                              (((0,), (0,)), ((), ())),
                              preferred_element_type=f32) + bo[...]
    y = out + xq
    mu = jnp.mean(y, axis=1, keepdims=True)
    yc = y - mu
    var = jnp.mean(yc * yc, axis=1, keepdims=True)
    out_ref[...] = yc * jax.lax.rsqrt(var + 1e-5) * gamma[...] + beta[...]


@jax.jit
def _attn_call(x, m, wq, bq, wk, bk, wv, bv, woT, bo, gamma, beta):
    full = lambda *_: (0, 0)
    specs = [
        pl.BlockSpec((BQ, D), lambda i: (i, 0)),
        pl.BlockSpec((N, D), full),
        pl.BlockSpec((N, N), full),
        pl.BlockSpec((D, D), full),
        pl.BlockSpec((D, 1), full),
        pl.BlockSpec((D, D), full),
        pl.BlockSpec((D, 1), full),
        pl.BlockSpec((D, D), full),
        pl.BlockSpec((D, 1), full),
        pl.BlockSpec((D, D), full),
        pl.BlockSpec((1, D), full),
        pl.BlockSpec((1, D), full),
        pl.BlockSpec((1, D), full),
    ]
    return pl.pallas_call(
        _attn_body,
        grid=(GRID,),
        in_specs=specs,
        out_specs=pl.BlockSpec((BQ, D), lambda i: (i, 0)),
        out_shape=jax.ShapeDtypeStruct((N, D), jnp.float32),
        scratch_shapes=[
            pltpu.VMEM((D, N), jnp.bfloat16),
            pltpu.VMEM((H * VR, N), jnp.bfloat16),
            pltpu.VMEM((N, N), jnp.float8_e4m3fn),
        ],
    )(x, x, m, wq, bq, wk, bk, wv, bv, woT, bo, gamma, beta)


E = 32768
NW = 32
RPB = 32
ECH = 16384


def _sc_scatter_body(edges, out_hbm, rows_v, sbuf, dbuf):
    wid = jax.lax.axis_index("s") * 2 + jax.lax.axis_index("c")
    lanes = jax.lax.iota(jnp.int32, 16)
    ones = jnp.ones((16,), jnp.float32)
    zeros = jnp.zeros((16,), jnp.float32)

    for blk in range(2):
        row_base = (wid * 2 + blk) * RPB

        def _zero(r, c):
            for k in range(N // 16):
                rows_v[r, pl.ds(k * 16, 16)] = zeros
            return c
        jax.lax.fori_loop(0, RPB, _zero, 0)

        for k in range(2):
            r = lanes + k * 16
            plsc.store_scatter(rows_v, [r, r + row_base], ones)

        for half in range(2):
            pltpu.sync_copy(edges.at[pl.ds(half * ECH, ECH)], sbuf)
            pltpu.sync_copy(edges.at[pl.ds(E + half * ECH, ECH)], dbuf)

            def _scan(j, c):
                s = sbuf[pl.ds(j * 16, 16)]
                d = dbuf[pl.ds(j * 16, 16)]
                rel = s - row_base
                own = (rel >= 0) & (rel < RPB)
                plsc.store_scatter(rows_v, [jnp.where(own, rel, 0), d],
                                   ones, mask=own)
                rel2 = d - row_base
                own2 = (rel2 >= 0) & (rel2 < RPB)
                plsc.store_scatter(rows_v, [jnp.where(own2, rel2, 0), s],
                                   ones, mask=own2)
                return c
            jax.lax.fori_loop(0, ECH // 16, _scan, 0)

        pltpu.sync_copy(rows_v, out_hbm.at[pl.ds(row_base, RPB), :])


@jax.jit
def _build_m(edge_index):
    edges = edge_index.reshape(2 * E)
    call = pl.kernel(
        _sc_scatter_body,
        out_type=jax.ShapeDtypeStruct((N, N), jnp.float32),
        mesh=plsc.VectorSubcoreMesh(core_axis_name="c", subcore_axis_name="s"),
        compiler_params=pltpu.CompilerParams(needs_layout_passes=False),
        scratch_types=[
            pltpu.VMEM((RPB, N), jnp.float32),
            pltpu.VMEM((ECH,), jnp.int32),
            pltpu.VMEM((ECH,), jnp.int32),
        ],
    )
    return call(edges)


def kernel(x, edge_index, Wq, bq, Wk, bk, Wv, bv, Wo, bo, gamma, beta):
    m = _build_m(edge_index.astype(jnp.int32))
    col = lambda b: b.reshape(D, 1)
    row = lambda b: b.reshape(1, D)
    return _attn_call(
        x, m,
        Wq.astype(jnp.bfloat16), col(bq),
        Wk.astype(jnp.bfloat16), col(bk),
        Wv.astype(jnp.bfloat16), col(bv),
        Wo.T.astype(jnp.bfloat16), row(bo),
        row(gamma), row(beta))


# speedup vs baseline: 3.9271x; 1.0897x over previous
import functools
import math

import jax
import jax.numpy as jnp
from jax.experimental import pallas as pl
from jax.experimental.pallas import tpu as pltpu
from jax.experimental.pallas import tpu_sc as plsc

N = 2048
D = 256
H = 8
HD = D // H
BQ = 256
GRID = N // BQ
RSCALE = 1.0 / math.sqrt(HD)
VR = 40


def _attn_body(x_blk, x_full, m_ref, wq, bq, wk, bk, wv, bv, woT, bo,
               gamma, beta, out_ref, kt_scr, vt_scr, m8):
    i = pl.program_id(0)
    f32 = jnp.float32
    bf16 = jnp.bfloat16

    f8 = jnp.float8_e4m3fn

    @pl.when(i == 0)
    def _():
        xb = x_full[...].astype(bf16)
        kt_scr[...] = (jax.lax.dot_general(
            wk[...], xb, (((1,), (1,)), ((), ())),
            preferred_element_type=f32) + bk[...]).astype(f8)
        vt = (jax.lax.dot_general(
            wv[...], xb, (((1,), (1,)), ((), ())),
            preferred_element_type=f32) + bv[...]).astype(f8)
        for h in range(H):
            vt_scr[h * VR:h * VR + HD, :] = vt[h * HD:(h + 1) * HD, :]
            vt_scr[h * VR + HD:h * VR + HD + 1, :] = jnp.ones((1, N), f8)
            vt_scr[h * VR + HD + 1:(h + 1) * VR, :] = (
                jnp.zeros((VR - HD - 1, N), f8))
        m8[...] = m_ref[...].astype(f8)

    xq = x_blk[...]
    qt = ((jax.lax.dot_general(wq[...], xq.astype(bf16),
                               (((1,), (1,)), ((), ())),
                               preferred_element_type=f32)
           + bq[...]) * RSCALE).astype(f8)

    mi = m8[pl.ds(i * BQ, BQ), :]
    reach = jax.lax.dot_general(mi, m8[...], (((1,), (0,)), ((), ())),
                                preferred_element_type=f32)
    nz16 = jnp.minimum(reach, 1.0).astype(bf16)

    att_rows = []
    for h in range(H):
        qh = qt[h * HD:(h + 1) * HD, :]
        kh = kt_scr[h * HD:(h + 1) * HD, :]
        s = jax.lax.dot_general(qh, kh, (((0,), (0,)), ((), ())),
                                preferred_element_type=f32)
        e8 = (jnp.exp(s.astype(bf16)) * nz16).astype(f8)
        vh = vt_scr[h * VR:h * VR + HD + 1, :]
        aT = jax.lax.dot_general(vh, e8, (((1,), (1,)), ((), ())),
                                 preferred_element_type=f32)
        att_rows.append(aT[:HD, :] / aT[HD:HD + 1, :])
    attT = jnp.concatenate(att_rows, axis=0)

    out = jax.lax.dot_general(attT.astype(bf16), woT[...],
                              (((0,), (0,)), ((), ())),
                              preferred_element_type=f32) + bo[...]
    y = out + xq
    mu = jnp.mean(y, axis=1, keepdims=True)
    yc = y - mu
    var = jnp.mean(yc * yc, axis=1, keepdims=True)
    out_ref[...] = yc * jax.lax.rsqrt(var + 1e-5) * gamma[...] + beta[...]


@jax.jit
def _attn_call(x, m, wq, bq, wk, bk, wv, bv, woT, bo, gamma, beta):
    full = lambda *_: (0, 0)
    specs = [
        pl.BlockSpec((BQ, D), lambda i: (i, 0)),
        pl.BlockSpec((N, D), full),
        pl.BlockSpec((N, N), full),
        pl.BlockSpec((D, D), full),
        pl.BlockSpec((D, 1), full),
        pl.BlockSpec((D, D), full),
        pl.BlockSpec((D, 1), full),
        pl.BlockSpec((D, D), full),
        pl.BlockSpec((D, 1), full),
        pl.BlockSpec((D, D), full),
        pl.BlockSpec((1, D), full),
        pl.BlockSpec((1, D), full),
        pl.BlockSpec((1, D), full),
    ]
    return pl.pallas_call(
        _attn_body,
        grid=(GRID,),
        in_specs=specs,
        out_specs=pl.BlockSpec((BQ, D), lambda i: (i, 0)),
        out_shape=jax.ShapeDtypeStruct((N, D), jnp.float32),
        scratch_shapes=[
            pltpu.VMEM((D, N), jnp.float8_e4m3fn),
            pltpu.VMEM((H * VR, N), jnp.float8_e4m3fn),
            pltpu.VMEM((N, N), jnp.float8_e4m3fn),
        ],
    )(x, x, m, wq, bq, wk, bk, wv, bv, woT, bo, gamma, beta)


E = 32768
NW = 32
RPB = 32
ECH = 16384


def _sc_scatter_body(edges, out_hbm, rows_v, sbuf, dbuf):
    wid = jax.lax.axis_index("s") * 2 + jax.lax.axis_index("c")
    lanes = jax.lax.iota(jnp.int32, 16)
    ones = jnp.ones((16,), jnp.float32)
    zeros = jnp.zeros((16,), jnp.float32)

    for blk in range(2):
        row_base = (wid * 2 + blk) * RPB

        def _zero(r, c):
            for k in range(N // 16):
                rows_v[r, pl.ds(k * 16, 16)] = zeros
            return c
        jax.lax.fori_loop(0, RPB, _zero, 0)

        for k in range(2):
            r = lanes + k * 16
            plsc.store_scatter(rows_v, [r, r + row_base], ones)

        for half in range(2):
            pltpu.sync_copy(edges.at[pl.ds(half * ECH, ECH)], sbuf)
            pltpu.sync_copy(edges.at[pl.ds(E + half * ECH, ECH)], dbuf)

            def _scan(j, c):
                s = sbuf[pl.ds(j * 16, 16)]
                d = dbuf[pl.ds(j * 16, 16)]
                rel = s - row_base
                own = (rel >= 0) & (rel < RPB)
                plsc.store_scatter(rows_v, [jnp.where(own, rel, 0), d],
                                   ones, mask=own)
                rel2 = d - row_base
                own2 = (rel2 >= 0) & (rel2 < RPB)
                plsc.store_scatter(rows_v, [jnp.where(own2, rel2, 0), s],
                                   ones, mask=own2)
                return c
            jax.lax.fori_loop(0, ECH // 16, _scan, 0)

        pltpu.sync_copy(rows_v, out_hbm.at[pl.ds(row_base, RPB), :])


@jax.jit
def _build_m(edge_index):
    edges = edge_index.reshape(2 * E)
    call = pl.kernel(
        _sc_scatter_body,
        out_type=jax.ShapeDtypeStruct((N, N), jnp.float32),
        mesh=plsc.VectorSubcoreMesh(core_axis_name="c", subcore_axis_name="s"),
        compiler_params=pltpu.CompilerParams(needs_layout_passes=False),
        scratch_types=[
            pltpu.VMEM((RPB, N), jnp.float32),
            pltpu.VMEM((ECH,), jnp.int32),
            pltpu.VMEM((ECH,), jnp.int32),
        ],
    )
    return call(edges)


def kernel(x, edge_index, Wq, bq, Wk, bk, Wv, bv, Wo, bo, gamma, beta):
    m = _build_m(edge_index.astype(jnp.int32))
    col = lambda b: b.reshape(D, 1)
    row = lambda b: b.reshape(1, D)
    return _attn_call(
        x, m,
        Wq.astype(jnp.bfloat16), col(bq),
        Wk.astype(jnp.bfloat16), col(bk),
        Wv.astype(jnp.bfloat16), col(bv),
        Wo.T.astype(jnp.bfloat16), row(bo),
        row(gamma), row(beta))
